# baseline (speedup 1.0000x reference)
import functools

import jax
import jax.numpy as jnp
from jax import lax
from jax.experimental import pallas as pl
from jax.experimental.pallas import tpu as pltpu
from jax.experimental.pallas import tpu_sc as plsc

N_ATOMS = 100000
N_BONDS = 255000
D_IN = 128
D_OUT = 128
BPAD = 16
NW = 32

DEG_A = [
    (1, 20000, 200),
    (2, 30000, 120),
    (3, 30000, 80),
    (4, 15000, 40),
    (5, 5000, 40),
]
DEG_B = [
    (1, 20000, 400),
    (2, 30000, 400),
    (3, 30000, 200),
    (4, 15000, 200),
    (5, 5000, 40),
]


def _gather_sum_body(deg_list, width, table_hbm, i1, i2, i3, i4, i5,
                     out_hbm, idxv0, idxv1, rows_v, outb_v,
                     sg0, sg1, so0, so1):
    sg = (sg0, sg1)
    so = (so0, so1)
    idx_v = (idxv0, idxv1)
    idx_refs = (i1, i2, i3, i4, i5)
    wid = lax.axis_index("s") * 2 + lax.axis_index("c")
    row_base = 0
    for d, n_d, C in deg_list:
        E = C * d
        K = n_d // C
        idx_hbm = idx_refs[d - 1]

        def start_chunk(k, b, E=E, C=C, idx_hbm=idx_hbm):
            e0 = pl.multiple_of(k * E, 8)
            pltpu.sync_copy(idx_hbm.at[pl.ds(e0, E)],
                            idx_v[b].at[pl.ds(0, E)])
            return pltpu.async_copy(
                table_hbm.at[idx_v[b].at[pl.ds(0, E)]],
                rows_v.at[b, pl.ds(0, E)], sg[b])

        def compute_chunk(b, d=d, C=C):
            @plsc.parallel_loop(0, C, step=1, unroll=8)
            def row_body(r):
                for v in range(width // 16):
                    s = rows_v[b, r * d, pl.ds(16 * v, 16)]
                    for j in range(1, d):
                        s = s + rows_v[b, r * d + j, pl.ds(16 * v, 16)]
                    outb_v[b, r, pl.ds(16 * v, 16)] = s

        def out_copy(k, b, d=d, C=C, row_base=row_base):
            r0 = pl.multiple_of(row_base + k * C, 8)
            src = rows_v.at[b, pl.ds(0, C)] if d == 1 \
                else outb_v.at[b, pl.ds(0, C)]
            return pltpu.async_copy(src, out_hbm.at[pl.ds(r0, C)], so[b])

        def drain_one(b, d=d, C=C):
            src = rows_v.at[b, pl.ds(0, C)] if d == 1 \
                else outb_v.at[b, pl.ds(0, C)]
            pltpu.make_async_copy(src, out_hbm.at[pl.ds(0, C)], so[b]).wait()

        nk = (K - 1 - wid) // NW + 1

        def pair_body(i, _, d=d):
            for b in (0, 1):
                k = 2 * i + b

                @pl.when(k < nk)
                def _main(k=k, b=b):
                    @pl.when(k + 1 < nk)
                    def _prefetch():
                        if d == 1:
                            @pl.when(k >= 1)
                            def _():
                                drain_one(1 - b)
                        start_chunk(wid + (k + 1) * NW, 1 - b)

                    pltpu.make_async_copy(
                        table_hbm.at[idx_v[b].at[pl.ds(0, E)]],
                        rows_v.at[b, pl.ds(0, E)], sg[b]).wait()
                    if d != 1:
                        @pl.when(k >= 2)
                        def _():
                            drain_one(b)
                        compute_chunk(b)
                    out_copy(wid + k * NW, b)

                @pl.when((k >= nk) & (k >= 2) & (k < nk + 2))
                def _drain(b=b):
                    drain_one(b)
            return 0

        start_chunk(wid, 0)
        lax.fori_loop(0, (nk + 3) // 2, pair_body, 0)
        row_base += n_d


@functools.cache
def _get_sc_kernels():
    mesh = plsc.VectorSubcoreMesh(
        core_axis_name="c", subcore_axis_name="s",
        num_cores=2, num_subcores=16)

    def build(deg_list, width, use_tc_tiling):
        emax = max(c * d for d, _, c in deg_list)
        cmax = max(c for d, _, c in deg_list if d > 1)
        body = functools.partial(_gather_sum_body, deg_list, width)
        return pl.kernel(
            body,
            out_type=jax.ShapeDtypeStruct((N_ATOMS, width), jnp.float32),
            mesh=mesh,
            scratch_types=[
                pltpu.VMEM((emax,), jnp.int32),
                pltpu.VMEM((emax,), jnp.int32),
                pltpu.VMEM((2, emax, width), jnp.float32),
                pltpu.VMEM((2, cmax, width), jnp.float32),
                pltpu.SemaphoreType.DMA,
                pltpu.SemaphoreType.DMA,
                pltpu.SemaphoreType.DMA,
                pltpu.SemaphoreType.DMA,
            ],
            compiler_params=pltpu.CompilerParams(
                use_tc_tiling_on_sc=use_tc_tiling),
        )

    atom_k = build(DEG_A, D_IN, True)
    bond_k = build(DEG_B, BPAD, False)
    return atom_k, bond_k


_BLK = 1000
_NBLK = N_ATOMS // _BLK
_DEG_BOUNDS = (20, 50, 80, 95)


def _deg_of_block(i):
    b = jnp.int32(0)
    for t in _DEG_BOUNDS:
        b = b + (i >= t).astype(jnp.int32)
    return b


def _p1_body(a_ref, b_ref, x_ref, waT_ref, wbT_ref, wsT_ref, bias_ref,
             t_ref, stats_ref):
    bf = jnp.bfloat16
    t = jnp.dot(x_ref[...].astype(bf), wsT_ref[...].astype(bf),
                preferred_element_type=jnp.float32)
    t = t + jnp.dot(a_ref[...].astype(bf), waT_ref[0].astype(bf),
                    preferred_element_type=jnp.float32)
    t = t + jnp.dot(b_ref[...].astype(bf), wbT_ref[0].astype(bf),
                    preferred_element_type=jnp.float32)
    t = jnp.maximum(t + bias_ref[...], 0.0)
    t_ref[...] = t

    @pl.when(pl.program_id(0) == 0)
    def _():
        stats_ref[...] = jnp.zeros_like(stats_ref)

    stats_ref[0:1, :] += jnp.sum(t, axis=0, keepdims=True)
    stats_ref[1:2, :] += jnp.sum(t * t, axis=0, keepdims=True)


_pass1 = pl.pallas_call(
    _p1_body,
    grid=(_NBLK,),
    in_specs=[
        pl.BlockSpec((_BLK, D_IN), lambda i: (i, 0)),
        pl.BlockSpec((_BLK, BPAD), lambda i: (i, 0)),
        pl.BlockSpec((_BLK, D_IN), lambda i: (i, 0)),
        pl.BlockSpec((1, D_IN, D_OUT), lambda i: (_deg_of_block(i), 0, 0)),
        pl.BlockSpec((1, BPAD, D_OUT), lambda i: (_deg_of_block(i), 0, 0)),
        pl.BlockSpec((D_IN, D_OUT), lambda i: (0, 0)),
        pl.BlockSpec((1, D_OUT), lambda i: (0, 0)),
    ],
    out_specs=[
        pl.BlockSpec((_BLK, D_OUT), lambda i: (i, 0)),
        pl.BlockSpec((8, D_OUT), lambda i: (0, 0)),
    ],
    out_shape=[
        jax.ShapeDtypeStruct((N_ATOMS, D_OUT), jnp.float32),
        jax.ShapeDtypeStruct((8, D_OUT), jnp.float32),
    ],
)


def _p2_body(t_ref, stats_ref, gamma_ref, beta_ref, o_ref):
    n = jnp.float32(N_ATOMS)
    mean = stats_ref[0:1, :] / n
    var = stats_ref[1:2, :] / n - mean * mean
    rstd = lax.rsqrt(var + 1e-5)
    o_ref[...] = (t_ref[...] - mean) * (rstd * gamma_ref[...]) + beta_ref[...]


_BLK2 = 2000

_pass2 = pl.pallas_call(
    _p2_body,
    grid=(N_ATOMS // _BLK2,),
    in_specs=[
        pl.BlockSpec((_BLK2, D_OUT), lambda i: (i, 0)),
        pl.BlockSpec((8, D_OUT), lambda i: (0, 0)),
        pl.BlockSpec((1, D_OUT), lambda i: (0, 0)),
        pl.BlockSpec((1, D_OUT), lambda i: (0, 0)),
    ],
    out_specs=pl.BlockSpec((_BLK2, D_OUT), lambda i: (i, 0)),
    out_shape=jax.ShapeDtypeStruct((N_ATOMS, D_OUT), jnp.float32),
)


def kernel(atom_features, bond_features,
           atom_neighbors_d1, bond_neighbors_d1,
           atom_neighbors_d2, bond_neighbors_d2,
           atom_neighbors_d3, bond_neighbors_d3,
           atom_neighbors_d4, bond_neighbors_d4,
           atom_neighbors_d5, bond_neighbors_d5,
           W_self, W_d1, W_d2, W_d3, W_d4, W_d5,
           bias, gamma, beta):
    an = [atom_neighbors_d1, atom_neighbors_d2, atom_neighbors_d3,
          atom_neighbors_d4, atom_neighbors_d5]
    bn = [bond_neighbors_d1, bond_neighbors_d2, bond_neighbors_d3,
          bond_neighbors_d4, bond_neighbors_d5]
    Ws = [W_d1, W_d2, W_d3, W_d4, W_d5]

    aidx = [a.reshape(-1).astype(jnp.int32) for a in an]
    bidx = [b.reshape(-1).astype(jnp.int32) for b in bn]
    bond_pad = jnp.pad(bond_features, ((0, 0), (0, BPAD - 6)))

    atom_k, bond_k = _get_sc_kernels()
    A = atom_k(atom_features, *aidx)
    B = bond_k(bond_pad, *bidx)

    waT = jnp.stack([W[:, :D_IN].T for W in Ws])
    wbT = jnp.stack([jnp.pad(W[:, D_IN:].T, ((0, BPAD - 6), (0, 0)))
                     for W in Ws])

    t, stats = _pass1(A, B, atom_features, waT, wbT, W_self.T,
                      bias.reshape(1, D_OUT))
    out = _pass2(t, stats, gamma.reshape(1, D_OUT), beta.reshape(1, D_OUT))
    return out


# speedup vs baseline: 1.0046x; 1.0046x over previous
import functools

import jax
import jax.numpy as jnp
from jax import lax
from jax.experimental import pallas as pl
from jax.experimental.pallas import tpu as pltpu
from jax.experimental.pallas import tpu_sc as plsc

N_ATOMS = 100000
N_BONDS = 255000
D_IN = 128
D_OUT = 128
BPAD = 16
NW = 32

DEG_A = [
    (1, 20000, 200),
    (2, 30000, 120),
    (3, 30000, 80),
    (4, 15000, 40),
    (5, 5000, 40),
]
DEG_B = [
    (1, 20000, 400),
    (2, 30000, 400),
    (3, 30000, 200),
    (4, 15000, 200),
    (5, 5000, 40),
]


def _gather_sum_body(deg_list, width, table_hbm, i1, i2, i3, i4, i5,
                     out_hbm, idxv0, idxv1, rows_v, outb_v,
                     sg0, sg1, so0, so1):
    sg = (sg0, sg1)
    so = (so0, so1)
    idx_v = (idxv0, idxv1)
    idx_refs = (i1, i2, i3, i4, i5)
    wid = lax.axis_index("s") * 2 + lax.axis_index("c")
    row_base = 0
    for d, n_d, C in deg_list:
        E = C * d
        K = n_d // C
        idx_hbm = idx_refs[d - 1]

        def start_chunk(k, b, E=E, C=C, idx_hbm=idx_hbm):
            e0 = pl.multiple_of(k * E, 8)
            pltpu.sync_copy(idx_hbm.at[pl.ds(e0, E)],
                            idx_v[b].at[pl.ds(0, E)])
            return pltpu.async_copy(
                table_hbm.at[idx_v[b].at[pl.ds(0, E)]],
                rows_v.at[b, pl.ds(0, E)], sg[b])

        def compute_chunk(b, d=d, C=C):
            @plsc.parallel_loop(0, C, step=1, unroll=4)
            def row_body(r):
                for v in range(width // 16):
                    s = rows_v[b, r * d, pl.ds(16 * v, 16)]
                    for j in range(1, d):
                        s = s + rows_v[b, r * d + j, pl.ds(16 * v, 16)]
                    outb_v[b, r, pl.ds(16 * v, 16)] = s

        def out_copy(k, b, d=d, C=C, row_base=row_base):
            r0 = pl.multiple_of(row_base + k * C, 8)
            src = rows_v.at[b, pl.ds(0, C)] if d == 1 \
                else outb_v.at[b, pl.ds(0, C)]
            return pltpu.async_copy(src, out_hbm.at[pl.ds(r0, C)], so[b])

        def drain_one(b, d=d, C=C):
            src = rows_v.at[b, pl.ds(0, C)] if d == 1 \
                else outb_v.at[b, pl.ds(0, C)]
            pltpu.make_async_copy(src, out_hbm.at[pl.ds(0, C)], so[b]).wait()

        nk = (K - 1 - wid) // NW + 1

        def pair_body(i, _, d=d):
            for b in (0, 1):
                k = 2 * i + b

                @pl.when(k < nk)
                def _main(k=k, b=b):
                    @pl.when(k + 1 < nk)
                    def _prefetch():
                        if d == 1:
                            @pl.when(k >= 1)
                            def _():
                                drain_one(1 - b)
                        start_chunk(wid + (k + 1) * NW, 1 - b)

                    pltpu.make_async_copy(
                        table_hbm.at[idx_v[b].at[pl.ds(0, E)]],
                        rows_v.at[b, pl.ds(0, E)], sg[b]).wait()
                    if d != 1:
                        @pl.when(k >= 2)
                        def _():
                            drain_one(b)
                        compute_chunk(b)
                    out_copy(wid + k * NW, b)

                @pl.when((k >= nk) & (k >= 2) & (k < nk + 2))
                def _drain(b=b):
                    drain_one(b)
            return 0

        start_chunk(wid, 0)
        lax.fori_loop(0, (nk + 3) // 2, pair_body, 0)
        row_base += n_d


@functools.cache
def _get_sc_kernels():
    mesh = plsc.VectorSubcoreMesh(
        core_axis_name="c", subcore_axis_name="s",
        num_cores=2, num_subcores=16)

    def build(deg_list, width, use_tc_tiling):
        emax = max(c * d for d, _, c in deg_list)
        cmax = max(c for d, _, c in deg_list if d > 1)
        body = functools.partial(_gather_sum_body, deg_list, width)
        return pl.kernel(
            body,
            out_type=jax.ShapeDtypeStruct((N_ATOMS, width), jnp.float32),
            mesh=mesh,
            scratch_types=[
                pltpu.VMEM((emax,), jnp.int32),
                pltpu.VMEM((emax,), jnp.int32),
                pltpu.VMEM((2, emax, width), jnp.float32),
                pltpu.VMEM((2, cmax, width), jnp.float32),
                pltpu.SemaphoreType.DMA,
                pltpu.SemaphoreType.DMA,
                pltpu.SemaphoreType.DMA,
                pltpu.SemaphoreType.DMA,
            ],
            compiler_params=pltpu.CompilerParams(
                use_tc_tiling_on_sc=use_tc_tiling),
        )

    atom_k = build(DEG_A, D_IN, True)
    bond_k = build(DEG_B, BPAD, False)
    return atom_k, bond_k


_BLK = 1000
_NBLK = N_ATOMS // _BLK
_DEG_BOUNDS = (20, 50, 80, 95)


def _deg_of_block(i):
    b = jnp.int32(0)
    for t in _DEG_BOUNDS:
        b = b + (i >= t).astype(jnp.int32)
    return b


def _p1_body(a_ref, b_ref, x_ref, waT_ref, wbT_ref, wsT_ref, bias_ref,
             t_ref, stats_ref):
    t = jnp.dot(x_ref[...], wsT_ref[...], preferred_element_type=jnp.float32)
    t = t + jnp.dot(a_ref[...], waT_ref[0], preferred_element_type=jnp.float32)
    t = t + jnp.dot(b_ref[...], wbT_ref[0], preferred_element_type=jnp.float32)
    t = jnp.maximum(t + bias_ref[...], 0.0)
    t_ref[...] = t

    @pl.when(pl.program_id(0) == 0)
    def _():
        stats_ref[...] = jnp.zeros_like(stats_ref)

    stats_ref[0:1, :] += jnp.sum(t, axis=0, keepdims=True)
    stats_ref[1:2, :] += jnp.sum(t * t, axis=0, keepdims=True)


_pass1 = pl.pallas_call(
    _p1_body,
    grid=(_NBLK,),
    in_specs=[
        pl.BlockSpec((_BLK, D_IN), lambda i: (i, 0)),
        pl.BlockSpec((_BLK, BPAD), lambda i: (i, 0)),
        pl.BlockSpec((_BLK, D_IN), lambda i: (i, 0)),
        pl.BlockSpec((1, D_IN, D_OUT), lambda i: (_deg_of_block(i), 0, 0)),
        pl.BlockSpec((1, BPAD, D_OUT), lambda i: (_deg_of_block(i), 0, 0)),
        pl.BlockSpec((D_IN, D_OUT), lambda i: (0, 0)),
        pl.BlockSpec((1, D_OUT), lambda i: (0, 0)),
    ],
    out_specs=[
        pl.BlockSpec((_BLK, D_OUT), lambda i: (i, 0)),
        pl.BlockSpec((8, D_OUT), lambda i: (0, 0)),
    ],
    out_shape=[
        jax.ShapeDtypeStruct((N_ATOMS, D_OUT), jnp.float32),
        jax.ShapeDtypeStruct((8, D_OUT), jnp.float32),
    ],
)


def _p2_body(t_ref, stats_ref, gamma_ref, beta_ref, o_ref):
    n = jnp.float32(N_ATOMS)
    mean = stats_ref[0:1, :] / n
    var = stats_ref[1:2, :] / n - mean * mean
    rstd = lax.rsqrt(var + 1e-5)
    o_ref[...] = (t_ref[...] - mean) * (rstd * gamma_ref[...]) + beta_ref[...]


_BLK2 = 2000

_pass2 = pl.pallas_call(
    _p2_body,
    grid=(N_ATOMS // _BLK2,),
    in_specs=[
        pl.BlockSpec((_BLK2, D_OUT), lambda i: (i, 0)),
        pl.BlockSpec((8, D_OUT), lambda i: (0, 0)),
        pl.BlockSpec((1, D_OUT), lambda i: (0, 0)),
        pl.BlockSpec((1, D_OUT), lambda i: (0, 0)),
    ],
    out_specs=pl.BlockSpec((_BLK2, D_OUT), lambda i: (i, 0)),
    out_shape=jax.ShapeDtypeStruct((N_ATOMS, D_OUT), jnp.float32),
)


def kernel(atom_features, bond_features,
           atom_neighbors_d1, bond_neighbors_d1,
           atom_neighbors_d2, bond_neighbors_d2,
           atom_neighbors_d3, bond_neighbors_d3,
           atom_neighbors_d4, bond_neighbors_d4,
           atom_neighbors_d5, bond_neighbors_d5,
           W_self, W_d1, W_d2, W_d3, W_d4, W_d5,
           bias, gamma, beta):
    an = [atom_neighbors_d1, atom_neighbors_d2, atom_neighbors_d3,
          atom_neighbors_d4, atom_neighbors_d5]
    bn = [bond_neighbors_d1, bond_neighbors_d2, bond_neighbors_d3,
          bond_neighbors_d4, bond_neighbors_d5]
    Ws = [W_d1, W_d2, W_d3, W_d4, W_d5]

    aidx = [a.reshape(-1).astype(jnp.int32) for a in an]
    bidx = [b.reshape(-1).astype(jnp.int32) for b in bn]
    bond_pad = jnp.pad(bond_features, ((0, 0), (0, BPAD - 6)))

    atom_k, bond_k = _get_sc_kernels()
    A = atom_k(atom_features, *aidx)
    B = bond_k(bond_pad, *bidx)

    waT = jnp.stack([W[:, :D_IN].T for W in Ws])
    wbT = jnp.stack([jnp.pad(W[:, D_IN:].T, ((0, BPAD - 6), (0, 0)))
                     for W in Ws])

    t, stats = _pass1(A, B, atom_features, waT, wbT, W_self.T,
                      bias.reshape(1, D_OUT))
    out = _pass2(t, stats, gamma.reshape(1, D_OUT), beta.reshape(1, D_OUT))
    return out


# speedup vs baseline: 1.0304x; 1.0256x over previous
import functools

import jax
import jax.numpy as jnp
from jax import lax
from jax.experimental import pallas as pl
from jax.experimental.pallas import tpu as pltpu
from jax.experimental.pallas import tpu_sc as plsc

N_ATOMS = 100000
N_BONDS = 255000
D_IN = 128
D_OUT = 128
BPAD = 16
NW = 32

DEG_A = [
    (1, 20000, 200),
    (2, 30000, 120),
    (3, 30000, 80),
    (4, 15000, 40),
    (5, 5000, 40),
]
DEG_B = [
    (1, 20000, 400),
    (2, 30000, 400),
    (3, 30000, 200),
    (4, 15000, 200),
    (5, 5000, 40),
]


def _gather_sum_body(deg_list, width, table_hbm, i1, i2, i3, i4, i5,
                     out_hbm, idxv0, idxv1, rows_v, outb_v,
                     sg0, sg1, so0, so1):
    sg = (sg0, sg1)
    so = (so0, so1)
    idx_v = (idxv0, idxv1)
    idx_refs = (i1, i2, i3, i4, i5)
    wid = lax.axis_index("s") * 2 + lax.axis_index("c")
    row_base = 0
    for d, n_d, C in deg_list:
        E = C * d
        K = n_d // C
        idx_hbm = idx_refs[d - 1]

        def start_chunk(k, b, E=E, C=C, idx_hbm=idx_hbm):
            e0 = pl.multiple_of(k * E, 8)
            pltpu.sync_copy(idx_hbm.at[pl.ds(e0, E)],
                            idx_v[b].at[pl.ds(0, E)])
            return pltpu.async_copy(
                table_hbm.at[idx_v[b].at[pl.ds(0, E)]],
                rows_v.at[b, pl.ds(0, E)], sg[b])

        def compute_chunk(b, d=d, C=C):
            @plsc.parallel_loop(0, C, step=1, unroll=4)
            def row_body(r):
                for v in range(width // 16):
                    s = rows_v[b, r * d, pl.ds(16 * v, 16)]
                    for j in range(1, d):
                        s = s + rows_v[b, r * d + j, pl.ds(16 * v, 16)]
                    outb_v[b, r, pl.ds(16 * v, 16)] = s

        def out_copy(k, b, d=d, C=C, row_base=row_base):
            r0 = pl.multiple_of(row_base + k * C, 8)
            src = rows_v.at[b, pl.ds(0, C)] if d == 1 \
                else outb_v.at[b, pl.ds(0, C)]
            return pltpu.async_copy(src, out_hbm.at[pl.ds(r0, C)], so[b])

        def drain_one(b, d=d, C=C):
            src = rows_v.at[b, pl.ds(0, C)] if d == 1 \
                else outb_v.at[b, pl.ds(0, C)]
            pltpu.make_async_copy(src, out_hbm.at[pl.ds(0, C)], so[b]).wait()

        nk = (K - 1 - wid) // NW + 1

        def pair_body(i, _, d=d):
            for b in (0, 1):
                k = 2 * i + b

                @pl.when(k < nk)
                def _main(k=k, b=b):
                    @pl.when(k + 1 < nk)
                    def _prefetch():
                        if d == 1:
                            @pl.when(k >= 1)
                            def _():
                                drain_one(1 - b)
                        start_chunk(wid + (k + 1) * NW, 1 - b)

                    pltpu.make_async_copy(
                        table_hbm.at[idx_v[b].at[pl.ds(0, E)]],
                        rows_v.at[b, pl.ds(0, E)], sg[b]).wait()
                    if d != 1:
                        @pl.when(k >= 2)
                        def _():
                            drain_one(b)
                        compute_chunk(b)
                    out_copy(wid + k * NW, b)

                @pl.when((k >= nk) & (k >= 2) & (k < nk + 2))
                def _drain(b=b):
                    drain_one(b)
            return 0

        start_chunk(wid, 0)
        lax.fori_loop(0, (nk + 3) // 2, pair_body, 0)
        row_base += n_d


@functools.cache
def _get_sc_kernels():
    mesh = plsc.VectorSubcoreMesh(
        core_axis_name="c", subcore_axis_name="s",
        num_cores=2, num_subcores=16)

    def build(deg_list, width, use_tc_tiling):
        emax = max(c * d for d, _, c in deg_list)
        cmax = max(c for d, _, c in deg_list if d > 1)
        body = functools.partial(_gather_sum_body, deg_list, width)
        return pl.kernel(
            body,
            out_type=jax.ShapeDtypeStruct((N_ATOMS, width), jnp.float32),
            mesh=mesh,
            scratch_types=[
                pltpu.VMEM((emax,), jnp.int32),
                pltpu.VMEM((emax,), jnp.int32),
                pltpu.VMEM((2, emax, width), jnp.float32),
                pltpu.VMEM((2, cmax, width), jnp.float32),
                pltpu.SemaphoreType.DMA,
                pltpu.SemaphoreType.DMA,
                pltpu.SemaphoreType.DMA,
                pltpu.SemaphoreType.DMA,
            ],
            compiler_params=pltpu.CompilerParams(
                use_tc_tiling_on_sc=use_tc_tiling),
        )

    atom_k = build(DEG_A, D_IN, True)
    bond_k = build(DEG_B, BPAD, False)
    return atom_k, bond_k


_BLK = 1000
_NBLK = N_ATOMS // _BLK
_DEG_BOUNDS = (20, 50, 80, 95)


def _deg_of_block(i):
    b = jnp.int32(0)
    for t in _DEG_BOUNDS:
        b = b + (i >= t).astype(jnp.int32)
    return b


def _p1_body(a_ref, b_ref, x_ref, waT_ref, wbT_ref, wsT_ref, bias_ref,
             t_ref, stats_ref):
    t = jnp.dot(x_ref[...], wsT_ref[...], preferred_element_type=jnp.float32)
    t = t + jnp.dot(a_ref[...], waT_ref[0], preferred_element_type=jnp.float32)
    t = t + jnp.dot(b_ref[...], wbT_ref[0], preferred_element_type=jnp.float32)
    t = jnp.maximum(t + bias_ref[...], 0.0)
    t_ref[...] = t

    @pl.when(pl.program_id(0) == 0)
    def _():
        stats_ref[...] = jnp.zeros_like(stats_ref)

    stats_ref[0:1, :] += jnp.sum(t, axis=0, keepdims=True)
    stats_ref[1:2, :] += jnp.sum(t * t, axis=0, keepdims=True)


_pass1 = pl.pallas_call(
    _p1_body,
    grid=(_NBLK,),
    in_specs=[
        pl.BlockSpec((_BLK, D_IN), lambda i: (i, 0)),
        pl.BlockSpec((_BLK, BPAD), lambda i: (i, 0)),
        pl.BlockSpec((_BLK, D_IN), lambda i: (i, 0)),
        pl.BlockSpec((1, D_IN, D_OUT), lambda i: (_deg_of_block(i), 0, 0)),
        pl.BlockSpec((1, BPAD, D_OUT), lambda i: (_deg_of_block(i), 0, 0)),
        pl.BlockSpec((D_IN, D_OUT), lambda i: (0, 0)),
        pl.BlockSpec((1, D_OUT), lambda i: (0, 0)),
    ],
    out_specs=[
        pl.BlockSpec((_BLK, D_OUT), lambda i: (i, 0)),
        pl.BlockSpec((8, D_OUT), lambda i: (0, 0)),
    ],
    out_shape=[
        jax.ShapeDtypeStruct((N_ATOMS, D_OUT), jnp.float32),
        jax.ShapeDtypeStruct((8, D_OUT), jnp.float32),
    ],
)


def _p2_body(t_ref, stats_ref, gamma_ref, beta_ref, o_ref):
    n = jnp.float32(N_ATOMS)
    mean = stats_ref[0:1, :] / n
    var = stats_ref[1:2, :] / n - mean * mean
    rstd = lax.rsqrt(var + 1e-5)
    o_ref[...] = (t_ref[...] - mean) * (rstd * gamma_ref[...]) + beta_ref[...]


_BLK2 = 4000

_pass2 = pl.pallas_call(
    _p2_body,
    grid=(N_ATOMS // _BLK2,),
    in_specs=[
        pl.BlockSpec((_BLK2, D_OUT), lambda i: (i, 0)),
        pl.BlockSpec((8, D_OUT), lambda i: (0, 0)),
        pl.BlockSpec((1, D_OUT), lambda i: (0, 0)),
        pl.BlockSpec((1, D_OUT), lambda i: (0, 0)),
    ],
    out_specs=pl.BlockSpec((_BLK2, D_OUT), lambda i: (i, 0)),
    out_shape=jax.ShapeDtypeStruct((N_ATOMS, D_OUT), jnp.float32),
)


def kernel(atom_features, bond_features,
           atom_neighbors_d1, bond_neighbors_d1,
           atom_neighbors_d2, bond_neighbors_d2,
           atom_neighbors_d3, bond_neighbors_d3,
           atom_neighbors_d4, bond_neighbors_d4,
           atom_neighbors_d5, bond_neighbors_d5,
           W_self, W_d1, W_d2, W_d3, W_d4, W_d5,
           bias, gamma, beta):
    an = [atom_neighbors_d1, atom_neighbors_d2, atom_neighbors_d3,
          atom_neighbors_d4, atom_neighbors_d5]
    bn = [bond_neighbors_d1, bond_neighbors_d2, bond_neighbors_d3,
          bond_neighbors_d4, bond_neighbors_d5]
    Ws = [W_d1, W_d2, W_d3, W_d4, W_d5]

    aidx = [a.reshape(-1).astype(jnp.int32) for a in an]
    bidx = [b.reshape(-1).astype(jnp.int32) for b in bn]
    bond_pad = jnp.pad(bond_features, ((0, 0), (0, BPAD - 6)))

    atom_k, bond_k = _get_sc_kernels()
    A = atom_k(atom_features, *aidx)
    B = bond_k(bond_pad, *bidx)

    waT = jnp.stack([W[:, :D_IN].T for W in Ws])
    wbT = jnp.stack([jnp.pad(W[:, D_IN:].T, ((0, BPAD - 6), (0, 0)))
                     for W in Ws])

    t, stats = _pass1(A, B, atom_features, waT, wbT, W_self.T,
                      bias.reshape(1, D_OUT))
    out = _pass2(t, stats, gamma.reshape(1, D_OUT), beta.reshape(1, D_OUT))
    return out


# speedup vs baseline: 1.0355x; 1.0049x over previous
import functools

import jax
import jax.numpy as jnp
from jax import lax
from jax.experimental import pallas as pl
from jax.experimental.pallas import tpu as pltpu
from jax.experimental.pallas import tpu_sc as plsc

N_ATOMS = 100000
N_BONDS = 255000
D_IN = 128
D_OUT = 128
BPAD = 16
NW = 32

DEG_A = [
    (1, 20000, 200),
    (2, 30000, 120),
    (3, 30000, 80),
    (4, 15000, 40),
    (5, 5000, 40),
]
DEG_B = [
    (1, 20000, 400),
    (2, 30000, 400),
    (3, 30000, 200),
    (4, 15000, 200),
    (5, 5000, 40),
]


def _gather_sum_body(deg_list, width, table_hbm, i1, i2, i3, i4, i5,
                     out_hbm, idxv0, idxv1, rows_v, outb_v,
                     sg0, sg1, so0, so1):
    sg = (sg0, sg1)
    so = (so0, so1)
    idx_v = (idxv0, idxv1)
    idx_refs = (i1, i2, i3, i4, i5)
    wid = lax.axis_index("s") * 2 + lax.axis_index("c")
    row_base = 0
    for d, n_d, C in deg_list:
        E = C * d
        K = n_d // C
        idx_hbm = idx_refs[d - 1]

        def start_chunk(k, b, E=E, C=C, idx_hbm=idx_hbm):
            e0 = pl.multiple_of(k * E, 8)
            pltpu.sync_copy(idx_hbm.at[pl.ds(e0, E)],
                            idx_v[b].at[pl.ds(0, E)])
            return pltpu.async_copy(
                table_hbm.at[idx_v[b].at[pl.ds(0, E)]],
                rows_v.at[b, pl.ds(0, E)], sg[b])

        def compute_chunk(b, d=d, C=C):
            @plsc.parallel_loop(0, C, step=1, unroll=4)
            def row_body(r):
                for v in range(width // 16):
                    s = rows_v[b, r * d, pl.ds(16 * v, 16)]
                    for j in range(1, d):
                        s = s + rows_v[b, r * d + j, pl.ds(16 * v, 16)]
                    outb_v[b, r, pl.ds(16 * v, 16)] = s

        def out_copy(k, b, d=d, C=C, row_base=row_base):
            r0 = pl.multiple_of(row_base + k * C, 8)
            src = rows_v.at[b, pl.ds(0, C)] if d == 1 \
                else outb_v.at[b, pl.ds(0, C)]
            return pltpu.async_copy(src, out_hbm.at[pl.ds(r0, C)], so[b])

        def drain_one(b, d=d, C=C):
            src = rows_v.at[b, pl.ds(0, C)] if d == 1 \
                else outb_v.at[b, pl.ds(0, C)]
            pltpu.make_async_copy(src, out_hbm.at[pl.ds(0, C)], so[b]).wait()

        nk = (K - 1 - wid) // NW + 1

        def pair_body(i, _, d=d):
            for b in (0, 1):
                k = 2 * i + b

                @pl.when(k < nk)
                def _main(k=k, b=b):
                    @pl.when(k + 1 < nk)
                    def _prefetch():
                        if d == 1:
                            @pl.when(k >= 1)
                            def _():
                                drain_one(1 - b)
                        start_chunk(wid + (k + 1) * NW, 1 - b)

                    pltpu.make_async_copy(
                        table_hbm.at[idx_v[b].at[pl.ds(0, E)]],
                        rows_v.at[b, pl.ds(0, E)], sg[b]).wait()
                    if d != 1:
                        @pl.when(k >= 2)
                        def _():
                            drain_one(b)
                        compute_chunk(b)
                    out_copy(wid + k * NW, b)

                @pl.when((k >= nk) & (k >= 2) & (k < nk + 2))
                def _drain(b=b):
                    drain_one(b)
            return 0

        start_chunk(wid, 0)
        lax.fori_loop(0, (nk + 3) // 2, pair_body, 0)
        row_base += n_d


@functools.cache
def _get_sc_kernels():
    mesh = plsc.VectorSubcoreMesh(
        core_axis_name="c", subcore_axis_name="s",
        num_cores=2, num_subcores=16)

    def build(deg_list, width, use_tc_tiling):
        emax = max(c * d for d, _, c in deg_list)
        cmax = max(c for d, _, c in deg_list if d > 1)
        body = functools.partial(_gather_sum_body, deg_list, width)
        return pl.kernel(
            body,
            out_type=jax.ShapeDtypeStruct((N_ATOMS, width), jnp.float32),
            mesh=mesh,
            scratch_types=[
                pltpu.VMEM((emax,), jnp.int32),
                pltpu.VMEM((emax,), jnp.int32),
                pltpu.VMEM((2, emax, width), jnp.float32),
                pltpu.VMEM((2, cmax, width), jnp.float32),
                pltpu.SemaphoreType.DMA,
                pltpu.SemaphoreType.DMA,
                pltpu.SemaphoreType.DMA,
                pltpu.SemaphoreType.DMA,
            ],
            compiler_params=pltpu.CompilerParams(
                use_tc_tiling_on_sc=use_tc_tiling),
        )

    atom_k = build(DEG_A, D_IN, True)
    bond_k = build(DEG_B, BPAD, False)
    return atom_k, bond_k


_BLK = 1000
_NBLK = N_ATOMS // _BLK
_DEG_BOUNDS = (20, 50, 80, 95)


def _deg_of_block(i):
    b = jnp.int32(0)
    for t in _DEG_BOUNDS:
        b = b + (i >= t).astype(jnp.int32)
    return b


def _p1_body(a_ref, b_ref, x_ref, waT_ref, wbT_ref, wsT_ref, bias_ref,
             t_ref, stats_ref):
    t = jnp.dot(x_ref[...], wsT_ref[...], preferred_element_type=jnp.float32)
    t = t + jnp.dot(a_ref[...], waT_ref[0], preferred_element_type=jnp.float32)
    t = t + jnp.dot(b_ref[...], wbT_ref[0], preferred_element_type=jnp.float32)
    t = jnp.maximum(t + bias_ref[...], 0.0)
    t_ref[...] = t

    @pl.when(pl.program_id(0) == 0)
    def _():
        stats_ref[...] = jnp.zeros_like(stats_ref)

    stats_ref[0:1, :] += jnp.sum(t, axis=0, keepdims=True)
    stats_ref[1:2, :] += jnp.sum(t * t, axis=0, keepdims=True)


_pass1 = pl.pallas_call(
    _p1_body,
    grid=(_NBLK,),
    in_specs=[
        pl.BlockSpec((_BLK, D_IN), lambda i: (i, 0)),
        pl.BlockSpec((_BLK, BPAD), lambda i: (i, 0)),
        pl.BlockSpec((_BLK, D_IN), lambda i: (i, 0)),
        pl.BlockSpec((1, D_IN, D_OUT), lambda i: (_deg_of_block(i), 0, 0)),
        pl.BlockSpec((1, BPAD, D_OUT), lambda i: (_deg_of_block(i), 0, 0)),
        pl.BlockSpec((D_IN, D_OUT), lambda i: (0, 0)),
        pl.BlockSpec((1, D_OUT), lambda i: (0, 0)),
    ],
    out_specs=[
        pl.BlockSpec((_BLK, D_OUT), lambda i: (i, 0)),
        pl.BlockSpec((8, D_OUT), lambda i: (0, 0)),
    ],
    out_shape=[
        jax.ShapeDtypeStruct((N_ATOMS, D_OUT), jnp.float32),
        jax.ShapeDtypeStruct((8, D_OUT), jnp.float32),
    ],
)


def _p2_body(t_ref, stats_ref, gamma_ref, beta_ref, o_ref):
    n = jnp.float32(N_ATOMS)
    mean = stats_ref[0:1, :] / n
    var = stats_ref[1:2, :] / n - mean * mean
    rstd = lax.rsqrt(var + 1e-5)
    o_ref[...] = (t_ref[...] - mean) * (rstd * gamma_ref[...]) + beta_ref[...]


_BLK2 = 10000

_pass2 = pl.pallas_call(
    _p2_body,
    grid=(N_ATOMS // _BLK2,),
    in_specs=[
        pl.BlockSpec((_BLK2, D_OUT), lambda i: (i, 0)),
        pl.BlockSpec((8, D_OUT), lambda i: (0, 0)),
        pl.BlockSpec((1, D_OUT), lambda i: (0, 0)),
        pl.BlockSpec((1, D_OUT), lambda i: (0, 0)),
    ],
    out_specs=pl.BlockSpec((_BLK2, D_OUT), lambda i: (i, 0)),
    out_shape=jax.ShapeDtypeStruct((N_ATOMS, D_OUT), jnp.float32),
)


def kernel(atom_features, bond_features,
           atom_neighbors_d1, bond_neighbors_d1,
           atom_neighbors_d2, bond_neighbors_d2,
           atom_neighbors_d3, bond_neighbors_d3,
           atom_neighbors_d4, bond_neighbors_d4,
           atom_neighbors_d5, bond_neighbors_d5,
           W_self, W_d1, W_d2, W_d3, W_d4, W_d5,
           bias, gamma, beta):
    an = [atom_neighbors_d1, atom_neighbors_d2, atom_neighbors_d3,
          atom_neighbors_d4, atom_neighbors_d5]
    bn = [bond_neighbors_d1, bond_neighbors_d2, bond_neighbors_d3,
          bond_neighbors_d4, bond_neighbors_d5]
    Ws = [W_d1, W_d2, W_d3, W_d4, W_d5]

    aidx = [a.reshape(-1).astype(jnp.int32) for a in an]
    bidx = [b.reshape(-1).astype(jnp.int32) for b in bn]
    bond_pad = jnp.pad(bond_features, ((0, 0), (0, BPAD - 6)))

    atom_k, bond_k = _get_sc_kernels()
    A = atom_k(atom_features, *aidx)
    B = bond_k(bond_pad, *bidx)

    waT = jnp.stack([W[:, :D_IN].T for W in Ws])
    wbT = jnp.stack([jnp.pad(W[:, D_IN:].T, ((0, BPAD - 6), (0, 0)))
                     for W in Ws])

    t, stats = _pass1(A, B, atom_features, waT, wbT, W_self.T,
                      bias.reshape(1, D_OUT))
    out = _pass2(t, stats, gamma.reshape(1, D_OUT), beta.reshape(1, D_OUT))
    return out
